# Initial kernel scaffold; baseline (speedup 1.0000x reference)
import jax
import jax.numpy as jnp
from jax.experimental import pallas as pl


def kernel(x, edge_index, batch, W11, b11, W12, b12, W21, b21, W22, b22, Wf, bf):
    raise NotImplementedError("write your pallas kernel here")


# speedup vs baseline: 1.3410x; 1.3410x over previous
import functools
import jax
import jax.numpy as jnp
from jax import lax
from jax.experimental import pallas as pl
from jax.experimental.pallas import tpu as pltpu

N_NODES = 50000
N_GRAPHS = 64
K_CHEB = 5


def _mm_relu_body(a_ref, w_ref, b_ref, o_ref):
    o_ref[...] = jax.nn.relu(
        jnp.dot(a_ref[...], w_ref[...], preferred_element_type=jnp.float32)
        + b_ref[...]
    )


def _matmul_relu(a, w, b, blk=2000):
    m, k = a.shape
    n = w.shape[1]
    return pl.pallas_call(
        _mm_relu_body,
        grid=(m // blk,),
        in_specs=[
            pl.BlockSpec((blk, k), lambda i: (i, 0)),
            pl.BlockSpec((k, n), lambda i: (0, 0)),
            pl.BlockSpec((1, n), lambda i: (0, 0)),
        ],
        out_specs=pl.BlockSpec((blk, n), lambda i: (i, 0)),
        out_shape=jax.ShapeDtypeStruct((m, n), jnp.float32),
    )(a, w, b.reshape(1, n))


def _pool_head_body(h_ref, b3_ref, wf_ref, bf_ref, o_ref, sum_ref, cnt_ref):
    i = pl.program_id(0)
    nb = pl.num_programs(0)

    @pl.when(i == 0)
    def _():
        sum_ref[...] = jnp.zeros_like(sum_ref)
        cnt_ref[...] = jnp.zeros_like(cnt_ref)

    bvec = b3_ref[0, 0, :]
    oh = (bvec[:, None] == lax.broadcasted_iota(jnp.int32, (1, N_GRAPHS), 1)
          ).astype(jnp.float32)
    sum_ref[...] += lax.dot_general(
        oh, h_ref[...], (((0,), (0,)), ((), ())),
        preferred_element_type=jnp.float32)
    cnt_ref[...] += jnp.sum(oh, axis=0, keepdims=True)

    @pl.when(i == nb - 1)
    def _():
        cnt = jnp.maximum(cnt_ref[0, :], 1.0)
        pooled = sum_ref[...] / cnt[:, None]
        logits = jnp.dot(pooled, wf_ref[...],
                         preferred_element_type=jnp.float32) + bf_ref[...]
        m = jnp.max(logits, axis=1, keepdims=True)
        e = jnp.exp(logits - m)
        o_ref[...] = (logits - m) - jnp.log(jnp.sum(e, axis=1, keepdims=True))


def _pool_head(h, batch, wf, bf, blk=1000):
    m, f = h.shape
    nb = m // blk
    b3 = batch.reshape(nb, 1, blk)
    return pl.pallas_call(
        _pool_head_body,
        grid=(nb,),
        in_specs=[
            pl.BlockSpec((blk, f), lambda i: (i, 0)),
            pl.BlockSpec((1, 1, blk), lambda i: (i, 0, 0)),
            pl.BlockSpec(wf.shape, lambda i: (0, 0)),
            pl.BlockSpec((1, wf.shape[1]), lambda i: (0, 0)),
        ],
        out_specs=pl.BlockSpec((N_GRAPHS, wf.shape[1]), lambda i: (0, 0)),
        out_shape=jax.ShapeDtypeStruct((N_GRAPHS, wf.shape[1]), jnp.float32),
        scratch_shapes=[
            pltpu.VMEM((N_GRAPHS, f), jnp.float32),
            pltpu.VMEM((1, N_GRAPHS), jnp.float32),
        ],
    )(h, b3, wf, bf.reshape(1, wf.shape[1]))


def _segsum(vals, idx, n):
    return jax.ops.segment_sum(vals, idx, num_segments=n)


def _cheb_stack(v0, gidx, sidx, dis, n):
    def lap(v):
        u = dis[:, None] * v
        return -dis[:, None] * _segsum(u[gidx], sidx, n)

    ts = [v0, lap(v0)]
    for _ in range(2, K_CHEB):
        ts.append(2.0 * lap(ts[-1]) - ts[-2])
    return ts


def kernel(x, edge_index, batch, W11, b11, W12, b12, W21, b21, W22, b22,
           Wf, bf):
    src = edge_index[0].astype(jnp.int32)
    dst = edge_index[1].astype(jnp.int32)
    n = N_NODES
    ones = jnp.ones((src.shape[0],), jnp.float32)
    deg_f = _segsum(ones, src, n)
    deg_r = _segsum(ones, dst, n)
    dis_f = jnp.where(deg_f > 0, lax.rsqrt(jnp.maximum(deg_f, 1.0)), 0.0)
    dis_r = jnp.where(deg_r > 0, lax.rsqrt(jnp.maximum(deg_r, 1.0)), 0.0)

    tf1 = _cheb_stack(x, src, dst, dis_f, n)
    tr1 = _cheb_stack(x, dst, src, dis_r, n)
    pf = jnp.concatenate(tf1, axis=1)
    pr = jnp.concatenate(tr1, axis=1)
    x1 = _matmul_relu(pf, W11.reshape(K_CHEB, 64), b11)
    x2 = _matmul_relu(pr, W12.reshape(K_CHEB, 64), b12)
    h = jnp.concatenate([x1, x2], axis=1)

    tf2 = _cheb_stack(h, src, dst, dis_f, n)
    tr2 = _cheb_stack(h, dst, src, dis_r, n)
    af = jnp.concatenate(tf2, axis=1)
    ar = jnp.concatenate(tr2, axis=1)
    x1 = _matmul_relu(af, W21.reshape(K_CHEB * 128, 256), b21)
    x2 = _matmul_relu(ar, W22.reshape(K_CHEB * 128, 256), b22)
    h2 = jnp.concatenate([x1, x2], axis=1)

    return _pool_head(h2, batch, Wf, bf)


# speedup vs baseline: 3.6153x; 2.6961x over previous
import functools
import jax
import jax.numpy as jnp
from jax import lax
from jax.experimental import pallas as pl
from jax.experimental.pallas import tpu as pltpu
from jax.experimental.pallas import tpu_sc as plsc

N_NODES = 50000
N_GRAPHS = 64
K_CHEB = 5
N_EDGES = 800000
_NSUB = 16
_EPT = N_EDGES // _NSUB
_EB = 80
_NPT = 3128
_NPAD = _NPT * _NSUB


def _make_sc_segsum(n_chunks, fc):
    C = n_chunks
    mesh = plsc.VectorSubcoreMesh(core_axis_name="c", subcore_axis_name="s")

    @functools.partial(
        pl.kernel,
        out_type=jax.ShapeDtypeStruct((2 * C * _NPAD, fc), jnp.float32),
        mesh=mesh,
        scratch_types=[
            pltpu.VMEM((_EB,), jnp.int32),
            pltpu.VMEM((_EB,), jnp.int32),
            pltpu.VMEM((_EB, fc), jnp.float32),
            pltpu.VMEM_SHARED((_NPAD, fc), jnp.float32),
            pltpu.SemaphoreType.DMA,
        ],
        compiler_params=pltpu.CompilerParams(use_tc_tiling_on_sc=False),
    )
    def segsum(u_hbm, gidx_hbm, sidx_hbm, zeros_hbm, out_hbm,
               gi_v, si_v, rows_v, acc_sh, sem):
        d = lax.axis_index("c")
        sid = lax.axis_index("s")
        for c in range(C):
            goff = (d * C + c) * _NPAD
            pltpu.sync_copy(zeros_hbm, acc_sh.at[pl.ds(sid * _NPT, _NPT)])
            plsc.subcore_barrier()

            def blk(b, carry):
                e0 = d * N_EDGES + sid * _EPT + b * _EB
                pltpu.sync_copy(gidx_hbm.at[pl.ds(e0, _EB)], gi_v)
                pltpu.sync_copy(sidx_hbm.at[pl.ds(e0, _EB)], si_v)
                for j in range(_EB // 16):
                    sl = pl.ds(j * 16, 16)
                    gi_v[sl] = gi_v[sl] + goff
                pltpu.async_copy(u_hbm.at[gi_v], rows_v, sem).wait()
                pltpu.sync_copy(rows_v, acc_sh.at[si_v], add=True)
                return carry

            lax.fori_loop(0, _EPT // _EB, blk, 0)
            plsc.subcore_barrier()
            pltpu.sync_copy(
                acc_sh.at[pl.ds(sid * _NPT, _NPT)],
                out_hbm.at[pl.ds(goff + sid * _NPT, _NPT)])
            plsc.subcore_barrier()

    return segsum


def _mm_relu_body(a_ref, w_ref, b_ref, o_ref):
    o_ref[...] = jax.nn.relu(
        jnp.dot(a_ref[...], w_ref[...], preferred_element_type=jnp.float32)
        + b_ref[...]
    )


def _matmul_relu(a, w, b, blk=2000):
    m, k = a.shape
    n = w.shape[1]
    return pl.pallas_call(
        _mm_relu_body,
        grid=(m // blk,),
        in_specs=[
            pl.BlockSpec((blk, k), lambda i: (i, 0)),
            pl.BlockSpec((k, n), lambda i: (0, 0)),
            pl.BlockSpec((1, n), lambda i: (0, 0)),
        ],
        out_specs=pl.BlockSpec((blk, n), lambda i: (i, 0)),
        out_shape=jax.ShapeDtypeStruct((m, n), jnp.float32),
    )(a, w, b.reshape(1, n))


def _pool_head_body(h_ref, b3_ref, wf_ref, bf_ref, o_ref, sum_ref, cnt_ref):
    i = pl.program_id(0)
    nb = pl.num_programs(0)

    @pl.when(i == 0)
    def _():
        sum_ref[...] = jnp.zeros_like(sum_ref)
        cnt_ref[...] = jnp.zeros_like(cnt_ref)

    bvec = b3_ref[0, 0, :]
    oh = (bvec[:, None] == lax.broadcasted_iota(jnp.int32, (1, N_GRAPHS), 1)
          ).astype(jnp.float32)
    sum_ref[...] += lax.dot_general(
        oh, h_ref[...], (((0,), (0,)), ((), ())),
        preferred_element_type=jnp.float32)
    cnt_ref[...] += jnp.sum(oh, axis=0, keepdims=True)

    @pl.when(i == nb - 1)
    def _():
        cnt = jnp.maximum(cnt_ref[0, :], 1.0)
        pooled = sum_ref[...] / cnt[:, None]
        logits = jnp.dot(pooled, wf_ref[...],
                         preferred_element_type=jnp.float32) + bf_ref[...]
        m = jnp.max(logits, axis=1, keepdims=True)
        e = jnp.exp(logits - m)
        o_ref[...] = (logits - m) - jnp.log(jnp.sum(e, axis=1, keepdims=True))


def _pool_head(h, batch, wf, bf, blk=1000):
    m, f = h.shape
    nb = m // blk
    b3 = batch.reshape(nb, 1, blk)
    return pl.pallas_call(
        _pool_head_body,
        grid=(nb,),
        in_specs=[
            pl.BlockSpec((blk, f), lambda i: (i, 0)),
            pl.BlockSpec((1, 1, blk), lambda i: (i, 0, 0)),
            pl.BlockSpec(wf.shape, lambda i: (0, 0)),
            pl.BlockSpec((1, wf.shape[1]), lambda i: (0, 0)),
        ],
        out_specs=pl.BlockSpec((N_GRAPHS, wf.shape[1]), lambda i: (0, 0)),
        out_shape=jax.ShapeDtypeStruct((N_GRAPHS, wf.shape[1]), jnp.float32),
        scratch_shapes=[
            pltpu.VMEM((N_GRAPHS, f), jnp.float32),
            pltpu.VMEM((1, N_GRAPHS), jnp.float32),
        ],
    )(h, b3, wf, bf.reshape(1, wf.shape[1]))


def _cheb_flat(t0_flat, dis_flat, segsum_call, gidx, sidx, zeros):
    ts = [t0_flat]
    u = dis_flat * t0_flat
    ts.append(-dis_flat * segsum_call(u, gidx, sidx, zeros))
    for _ in range(2, K_CHEB):
        u = dis_flat * ts[-1]
        ts.append(-2.0 * dis_flat * segsum_call(u, gidx, sidx, zeros)
                  - ts[-2])
    return ts


def kernel(x, edge_index, batch, W11, b11, W12, b12, W21, b21, W22, b22,
           Wf, bf):
    n = N_NODES
    src = edge_index[0].astype(jnp.int32)
    dst = edge_index[1].astype(jnp.int32)
    gidx = jnp.concatenate([src, dst])
    sidx = jnp.concatenate([dst, src])

    seg16 = _make_sc_segsum(1, 16)
    seg32 = _make_sc_segsum(4, 32)
    z16 = jnp.zeros((_NPT, 16), jnp.float32)
    z32 = jnp.zeros((_NPT, 32), jnp.float32)

    npad = _NPAD - n

    degs = seg16(jnp.ones((2 * _NPAD, 16), jnp.float32), gidx, sidx, z16)
    deg_r = degs[:n, 0]
    deg_f = degs[_NPAD:_NPAD + n, 0]
    dis_f = jnp.where(deg_f > 0, lax.rsqrt(jnp.maximum(deg_f, 1.0)), 0.0)
    dis_r = jnp.where(deg_r > 0, lax.rsqrt(jnp.maximum(deg_r, 1.0)), 0.0)
    dis2 = jnp.pad(jnp.stack([dis_f, dis_r]), ((0, 0), (0, npad)))
    dis1 = dis2.reshape(-1, 1)

    x16 = jnp.pad(x, ((0, npad), (0, 15)))
    t0 = jnp.concatenate([x16, x16])
    ts1 = _cheb_flat(t0, dis1, seg16, gidx, sidx, z16)
    pf = jnp.stack([t[:n, 0] for t in ts1], axis=1)
    pr = jnp.stack([t[_NPAD:_NPAD + n, 0] for t in ts1], axis=1)
    x1 = _matmul_relu(pf, W11.reshape(K_CHEB, 64), b11)
    x2 = _matmul_relu(pr, W12.reshape(K_CHEB, 64), b12)
    h = jnp.concatenate([x1, x2], axis=1)

    def to_flat(v2):
        return v2.reshape(2, _NPAD, 4, 32).transpose(0, 2, 1, 3).reshape(-1, 32)

    def from_flat(f):
        return f.reshape(2, 4, _NPAD, 32)[:, :, :n].transpose(
            0, 2, 1, 3).reshape(2, n, 128)

    disl2 = jnp.broadcast_to(dis2[:, None, :], (2, 4, _NPAD)).reshape(-1, 1)
    hp = jnp.pad(h, ((0, npad), (0, 0)))
    t0l2 = to_flat(jnp.stack([hp, hp]))
    ts2 = _cheb_flat(t0l2, disl2, seg32, gidx, sidx, z32)
    ts2 = [from_flat(t) for t in ts2]
    af = jnp.concatenate([t[0] for t in ts2], axis=1)
    ar = jnp.concatenate([t[1] for t in ts2], axis=1)
    x1 = _matmul_relu(af, W21.reshape(K_CHEB * 128, 256), b21)
    x2 = _matmul_relu(ar, W22.reshape(K_CHEB * 128, 256), b22)
    h2 = jnp.concatenate([x1, x2], axis=1)

    return _pool_head(h2, batch, Wf, bf)


# speedup vs baseline: 11.0304x; 3.0510x over previous
import functools
import jax
import jax.numpy as jnp
from jax import lax
from jax.experimental import pallas as pl
from jax.experimental.pallas import tpu as pltpu
from jax.experimental.pallas import tpu_sc as plsc

N_NODES = 50000
N_GRAPHS = 64
K_CHEB = 5
N_EDGES = 800000
_NSUB = 16
_EPT = N_EDGES // _NSUB
_EB = 80
_K = 10
_SB = _K * _EB
_NPT = 3128
_NPAD = _NPT * _NSUB


def _make_sc_segsum(n_chunks, fc):
    C = n_chunks
    mesh = plsc.VectorSubcoreMesh(core_axis_name="c", subcore_axis_name="s")

    @functools.partial(
        pl.kernel,
        out_type=jax.ShapeDtypeStruct((2 * C * _NPAD, fc), jnp.float32),
        mesh=mesh,
        scratch_types=[
            pltpu.VMEM((_SB,), jnp.int32),
            pltpu.VMEM((_K, _EB), jnp.int32),
            pltpu.VMEM((_K, _EB, fc), jnp.float32),
            pltpu.VMEM_SHARED((_NPAD, fc), jnp.float32),
            pltpu.SemaphoreType.DMA,
            pltpu.SemaphoreType.DMA,
        ],
        compiler_params=pltpu.CompilerParams(use_tc_tiling_on_sc=False),
    )
    def segsum(u_hbm, gidx_hbm, sidx2d_hbm, zeros_hbm, out_hbm,
               gi_v, si_v, rows_v, acc_sh, gsem, ssem):
        d = lax.axis_index("c")
        sid = lax.axis_index("s")
        for c in range(C):
            goff = (d * C + c) * _NPAD
            pltpu.sync_copy(zeros_hbm, acc_sh.at[pl.ds(sid * _NPT, _NPT)])
            plsc.subcore_barrier()

            def sblk(s, carry):
                e0 = d * N_EDGES + sid * _EPT + s * _SB
                pltpu.sync_copy(gidx_hbm.at[pl.ds(e0, _SB)], gi_v)
                pltpu.sync_copy(sidx2d_hbm.at[pl.ds(e0 // _EB, _K)], si_v)
                for j in range(_SB // 16):
                    sl = pl.ds(j * 16, 16)
                    gi_v[sl] = gi_v[sl] + goff
                gets = [
                    pltpu.async_copy(
                        u_hbm.at[gi_v.at[pl.ds(k * _EB, _EB)]],
                        rows_v.at[k], gsem)
                    for k in range(_K)
                ]
                puts = []
                for k in range(_K):
                    gets[k].wait()
                    puts.append(pltpu.async_copy(
                        rows_v.at[k], acc_sh.at[si_v.at[k]], ssem,
                        add=True))
                for p in puts:
                    p.wait()
                return carry

            lax.fori_loop(0, _EPT // _SB, sblk, 0)
            plsc.subcore_barrier()
            pltpu.sync_copy(
                acc_sh.at[pl.ds(sid * _NPT, _NPT)],
                out_hbm.at[pl.ds(goff + sid * _NPT, _NPT)])
            plsc.subcore_barrier()

    return segsum


def _mm_relu_body(a_ref, w_ref, b_ref, o_ref):
    o_ref[...] = jax.nn.relu(
        jnp.dot(a_ref[...], w_ref[...], preferred_element_type=jnp.float32)
        + b_ref[...]
    )


def _matmul_relu(a, w, b, blk=2000):
    m, k = a.shape
    n = w.shape[1]
    return pl.pallas_call(
        _mm_relu_body,
        grid=(m // blk,),
        in_specs=[
            pl.BlockSpec((blk, k), lambda i: (i, 0)),
            pl.BlockSpec((k, n), lambda i: (0, 0)),
            pl.BlockSpec((1, n), lambda i: (0, 0)),
        ],
        out_specs=pl.BlockSpec((blk, n), lambda i: (i, 0)),
        out_shape=jax.ShapeDtypeStruct((m, n), jnp.float32),
    )(a, w, b.reshape(1, n))


def _pool_head_body(h_ref, b3_ref, wf_ref, bf_ref, o_ref, sum_ref, cnt_ref):
    i = pl.program_id(0)
    nb = pl.num_programs(0)

    @pl.when(i == 0)
    def _():
        sum_ref[...] = jnp.zeros_like(sum_ref)
        cnt_ref[...] = jnp.zeros_like(cnt_ref)

    bvec = b3_ref[0, 0, :]
    oh = (bvec[:, None] == lax.broadcasted_iota(jnp.int32, (1, N_GRAPHS), 1)
          ).astype(jnp.float32)
    sum_ref[...] += lax.dot_general(
        oh, h_ref[...], (((0,), (0,)), ((), ())),
        preferred_element_type=jnp.float32)
    cnt_ref[...] += jnp.sum(oh, axis=0, keepdims=True)

    @pl.when(i == nb - 1)
    def _():
        cnt = jnp.maximum(cnt_ref[0, :], 1.0)
        pooled = sum_ref[...] / cnt[:, None]
        logits = jnp.dot(pooled, wf_ref[...],
                         preferred_element_type=jnp.float32) + bf_ref[...]
        m = jnp.max(logits, axis=1, keepdims=True)
        e = jnp.exp(logits - m)
        o_ref[...] = (logits - m) - jnp.log(jnp.sum(e, axis=1, keepdims=True))


def _pool_head(h, batch, wf, bf, blk=1000):
    m, f = h.shape
    nb = m // blk
    b3 = batch.reshape(nb, 1, blk)
    return pl.pallas_call(
        _pool_head_body,
        grid=(nb,),
        in_specs=[
            pl.BlockSpec((blk, f), lambda i: (i, 0)),
            pl.BlockSpec((1, 1, blk), lambda i: (i, 0, 0)),
            pl.BlockSpec(wf.shape, lambda i: (0, 0)),
            pl.BlockSpec((1, wf.shape[1]), lambda i: (0, 0)),
        ],
        out_specs=pl.BlockSpec((N_GRAPHS, wf.shape[1]), lambda i: (0, 0)),
        out_shape=jax.ShapeDtypeStruct((N_GRAPHS, wf.shape[1]), jnp.float32),
        scratch_shapes=[
            pltpu.VMEM((N_GRAPHS, f), jnp.float32),
            pltpu.VMEM((1, N_GRAPHS), jnp.float32),
        ],
    )(h, b3, wf, bf.reshape(1, wf.shape[1]))


def _cheb_flat(t0_flat, dis_flat, segsum_call, gidx, sidx, zeros):
    ts = [t0_flat]
    u = dis_flat * t0_flat
    ts.append(-dis_flat * segsum_call(u, gidx, sidx, zeros))
    for _ in range(2, K_CHEB):
        u = dis_flat * ts[-1]
        ts.append(-2.0 * dis_flat * segsum_call(u, gidx, sidx, zeros)
                  - ts[-2])
    return ts


def kernel(x, edge_index, batch, W11, b11, W12, b12, W21, b21, W22, b22,
           Wf, bf):
    n = N_NODES
    src = edge_index[0].astype(jnp.int32)
    dst = edge_index[1].astype(jnp.int32)
    gidx = jnp.concatenate([src, dst])
    sidx = jnp.concatenate([dst, src]).reshape(-1, _EB)

    seg16 = _make_sc_segsum(1, 16)
    seg32 = _make_sc_segsum(4, 32)
    z16 = jnp.zeros((_NPT, 16), jnp.float32)
    z32 = jnp.zeros((_NPT, 32), jnp.float32)

    npad = _NPAD - n

    degs = seg16(jnp.ones((2 * _NPAD, 16), jnp.float32), gidx, sidx, z16)
    deg_r = degs[:n, 0]
    deg_f = degs[_NPAD:_NPAD + n, 0]
    dis_f = jnp.where(deg_f > 0, lax.rsqrt(jnp.maximum(deg_f, 1.0)), 0.0)
    dis_r = jnp.where(deg_r > 0, lax.rsqrt(jnp.maximum(deg_r, 1.0)), 0.0)
    dis2 = jnp.pad(jnp.stack([dis_f, dis_r]), ((0, 0), (0, npad)))
    dis1 = dis2.reshape(-1, 1)

    x16 = jnp.pad(x, ((0, npad), (0, 15)))
    t0 = jnp.concatenate([x16, x16])
    ts1 = _cheb_flat(t0, dis1, seg16, gidx, sidx, z16)
    pf = jnp.stack([t[:n, 0] for t in ts1], axis=1)
    pr = jnp.stack([t[_NPAD:_NPAD + n, 0] for t in ts1], axis=1)
    x1 = _matmul_relu(pf, W11.reshape(K_CHEB, 64), b11)
    x2 = _matmul_relu(pr, W12.reshape(K_CHEB, 64), b12)
    h = jnp.concatenate([x1, x2], axis=1)

    def to_flat(v2):
        return v2.reshape(2, _NPAD, 4, 32).transpose(0, 2, 1, 3).reshape(-1, 32)

    def from_flat(f):
        return f.reshape(2, 4, _NPAD, 32)[:, :, :n].transpose(
            0, 2, 1, 3).reshape(2, n, 128)

    disl2 = jnp.broadcast_to(dis2[:, None, :], (2, 4, _NPAD)).reshape(-1, 1)
    hp = jnp.pad(h, ((0, npad), (0, 0)))
    t0l2 = to_flat(jnp.stack([hp, hp]))
    ts2 = _cheb_flat(t0l2, disl2, seg32, gidx, sidx, z32)
    ts2 = [from_flat(t) for t in ts2]
    af = jnp.concatenate([t[0] for t in ts2], axis=1)
    ar = jnp.concatenate([t[1] for t in ts2], axis=1)
    x1 = _matmul_relu(af, W21.reshape(K_CHEB * 128, 256), b21)
    x2 = _matmul_relu(ar, W22.reshape(K_CHEB * 128, 256), b22)
    h2 = jnp.concatenate([x1, x2], axis=1)

    return _pool_head(h2, batch, Wf, bf)
